# baseline (speedup 1.0000x reference)
import functools

import jax
import jax.numpy as jnp
import numpy as np
from jax import lax
from jax.experimental import pallas as pl
from jax.experimental.pallas import tpu as pltpu
from jax.experimental.pallas import tpu_sc as plsc

B = 1024
T = 441
D = 3060 * 3

NC, NS, L = 2, 16, 16
NW = NC * NS
BPW = B // NW
CH = 8
HALF_PI = np.float32(np.pi / 2)

_mesh = plsc.VectorSubcoreMesh(core_axis_name="c", subcore_axis_name="s")


@functools.partial(
    pl.kernel,
    out_type=(
        jax.ShapeDtypeStruct((B, D), jnp.int32),
        jax.ShapeDtypeStruct((B, D), jnp.int32),
        jax.ShapeDtypeStruct((B,), jnp.int32),
    ),
    mesh=_mesh,
    scratch_types=[
        pltpu.VMEM((BPW,), jnp.float32),
        pltpu.VMEM((BPW,), jnp.float32),
        pltpu.VMEM((128,), jnp.float32),
        pltpu.VMEM((BPW,), jnp.int32),
        pltpu.VMEM((CH, D), jnp.int32),
        pltpu.SemaphoreType.DMA,
    ],
)
def _lookup(yaw_hbm, pitch_hbm, params_hbm, vids_hbm, wets_hbm,
            vids_out, wets_out, idx_out,
            yaw_v, pitch_v, params_v, idx_v, buf, sem):
    wid = lax.axis_index("s") * NC + lax.axis_index("c")
    base = wid * BPW

    pltpu.sync_copy(yaw_hbm.at[pl.ds(base, BPW)], yaw_v)
    pltpu.sync_copy(pitch_hbm.at[pl.ds(base, BPW)], pitch_v)
    pltpu.sync_copy(params_hbm, params_v)

    minx = params_v[pl.ds(0, L)]
    maxx = params_v[pl.ds(16, L)]
    intx = params_v[pl.ds(32, L)]
    miny = params_v[pl.ds(48, L)]
    maxy = params_v[pl.ds(64, L)]
    inty = params_v[pl.ds(80, L)]
    nx_i = params_v[pl.ds(96, L)].astype(jnp.int32)

    for j in range(BPW // L):
        yv = yaw_v[pl.ds(j * L, L)]
        pv = pitch_v[pl.ds(j * L, L)]
        y = yv - HALF_PI
        p = HALF_PI - pv
        xc = jnp.clip(y, minx, maxx)
        x_id = ((xc - minx) / intx + np.float32(0.5)).astype(jnp.int32)
        yc = jnp.clip(p, miny, maxy)
        y_id = ((yc - miny) / inty + np.float32(0.5)).astype(jnp.int32)
        idx_v[pl.ds(j * L, L)] = y_id * nx_i + x_id

    pltpu.sync_copy(idx_v, idx_out.at[pl.ds(base, BPW)])

    idx_vecs = [idx_v[pl.ds(j * L, L)] for j in range(BPW // L)]
    for r in range(BPW):
        i = idx_vecs[r // L][r % L]
        pltpu.async_copy(vids_hbm.at[i], vids_out.at[base + r], sem).wait()
        pltpu.async_copy(wets_hbm.at[i], wets_out.at[base + r], sem).wait()


def kernel(yaw, pitch, n_y_p, min_v, max_v, vids, wets):
    vids2 = vids.reshape(T, D)
    wets2 = wets.reshape(T, D).view(jnp.int32)
    interval = (max_v - min_v) / (n_y_p - 1.0)
    params = jnp.concatenate([
        jnp.broadcast_to(min_v[0], (16,)),
        jnp.broadcast_to(max_v[0], (16,)),
        jnp.broadcast_to(interval[0], (16,)),
        jnp.broadcast_to(min_v[1], (16,)),
        jnp.broadcast_to(max_v[1], (16,)),
        jnp.broadcast_to(interval[1], (16,)),
        jnp.broadcast_to(n_y_p[0], (16,)),
        jnp.zeros((16,), jnp.float32),
    ])
    vids_rows, wets_rows, idx = _lookup(yaw, pitch, params, vids2, wets2)
    return (vids_rows.reshape(B, 3060, 3),
            wets_rows.view(jnp.float32).reshape(B, 3060, 3),
            idx)


# speedup vs baseline: 1.0005x; 1.0005x over previous
import functools

import jax
import jax.numpy as jnp
import numpy as np
from jax import lax
from jax.experimental import pallas as pl
from jax.experimental.pallas import tpu as pltpu
from jax.experimental.pallas import tpu_sc as plsc

B = 1024
T = 441
D = 3060 * 3

NC, NS, L = 2, 16, 16
NW = NC * NS
BPW = B // NW
CH = 8
HALF_PI = np.float32(np.pi / 2)

_mesh = plsc.VectorSubcoreMesh(core_axis_name="c", subcore_axis_name="s")


@functools.partial(
    pl.kernel,
    out_type=(
        jax.ShapeDtypeStruct((B, D), jnp.int32),
        jax.ShapeDtypeStruct((B, D), jnp.int32),
        jax.ShapeDtypeStruct((B,), jnp.int32),
    ),
    mesh=_mesh,
    scratch_types=[
        pltpu.VMEM((BPW,), jnp.float32),
        pltpu.VMEM((BPW,), jnp.float32),
        pltpu.VMEM((128,), jnp.float32),
        pltpu.VMEM((BPW,), jnp.int32),
        pltpu.VMEM((CH, D), jnp.int32),
        pltpu.SemaphoreType.DMA,
    ],
)
def _lookup(yaw_hbm, pitch_hbm, params_hbm, vids_hbm, wets_hbm,
            vids_out, wets_out, idx_out,
            yaw_v, pitch_v, params_v, idx_v, buf, sem):
    wid = lax.axis_index("s") * NC + lax.axis_index("c")
    base = wid * BPW

    pltpu.sync_copy(yaw_hbm.at[pl.ds(base, BPW)], yaw_v)
    pltpu.sync_copy(pitch_hbm.at[pl.ds(base, BPW)], pitch_v)
    pltpu.sync_copy(params_hbm, params_v)

    minx = params_v[pl.ds(0, L)]
    maxx = params_v[pl.ds(16, L)]
    intx = params_v[pl.ds(32, L)]
    miny = params_v[pl.ds(48, L)]
    maxy = params_v[pl.ds(64, L)]
    inty = params_v[pl.ds(80, L)]
    nx_i = params_v[pl.ds(96, L)].astype(jnp.int32)

    for j in range(BPW // L):
        yv = yaw_v[pl.ds(j * L, L)]
        pv = pitch_v[pl.ds(j * L, L)]
        y = yv - HALF_PI
        p = HALF_PI - pv
        xc = jnp.clip(y, minx, maxx)
        x_id = ((xc - minx) / intx + np.float32(0.5)).astype(jnp.int32)
        yc = jnp.clip(p, miny, maxy)
        y_id = ((yc - miny) / inty + np.float32(0.5)).astype(jnp.int32)
        idx_v[pl.ds(j * L, L)] = y_id * nx_i + x_id

    pltpu.sync_copy(idx_v, idx_out.at[pl.ds(base, BPW)])

    idx_vecs = [idx_v[pl.ds(j * L, L)] for j in range(BPW // L)]
    copies = []
    for r in range(BPW):
        i = idx_vecs[r // L][r % L]
        copies.append(
            pltpu.async_copy(vids_hbm.at[i], vids_out.at[base + r], sem))
        copies.append(
            pltpu.async_copy(wets_hbm.at[i], wets_out.at[base + r], sem))
    for c in copies:
        c.wait()


def kernel(yaw, pitch, n_y_p, min_v, max_v, vids, wets):
    vids2 = vids.reshape(T, D)
    wets2 = wets.reshape(T, D).view(jnp.int32)
    interval = (max_v - min_v) / (n_y_p - 1.0)
    params = jnp.concatenate([
        jnp.broadcast_to(min_v[0], (16,)),
        jnp.broadcast_to(max_v[0], (16,)),
        jnp.broadcast_to(interval[0], (16,)),
        jnp.broadcast_to(min_v[1], (16,)),
        jnp.broadcast_to(max_v[1], (16,)),
        jnp.broadcast_to(interval[1], (16,)),
        jnp.broadcast_to(n_y_p[0], (16,)),
        jnp.zeros((16,), jnp.float32),
    ])
    vids_rows, wets_rows, idx = _lookup(yaw, pitch, params, vids2, wets2)
    return (vids_rows.reshape(B, 3060, 3),
            wets_rows.view(jnp.float32).reshape(B, 3060, 3),
            idx)


# speedup vs baseline: 4.0518x; 4.0496x over previous
import functools

import jax
import jax.numpy as jnp
import numpy as np
from jax import lax
from jax.experimental import pallas as pl
from jax.experimental.pallas import tpu as pltpu
from jax.experimental.pallas import tpu_sc as plsc

B = 1024
T = 441
D = 3060 * 3

NC, NS, L = 2, 16, 16
NW = NC * NS
BPW = B // NW
CH = 8
HALF_PI = np.float32(np.pi / 2)

_mesh = plsc.VectorSubcoreMesh(core_axis_name="c", subcore_axis_name="s")


@functools.partial(
    pl.kernel,
    out_type=(
        jax.ShapeDtypeStruct((B, D), jnp.int32),
        jax.ShapeDtypeStruct((B, D), jnp.int32),
        jax.ShapeDtypeStruct((B,), jnp.int32),
    ),
    mesh=_mesh,
    scratch_types=[
        pltpu.VMEM((BPW,), jnp.float32),
        pltpu.VMEM((BPW,), jnp.float32),
        pltpu.VMEM((128,), jnp.float32),
        pltpu.VMEM((BPW,), jnp.int32),
        pltpu.VMEM((CH, D), jnp.int32),
        pltpu.SemaphoreType.DMA,
        pltpu.SemaphoreType.DMA,
    ],
)
def _lookup(yaw_hbm, pitch_hbm, params_hbm, vids_hbm, wets_hbm,
            vids_out, wets_out, idx_out,
            yaw_v, pitch_v, params_v, idx_v, buf, gsem, wsem):
    wid = lax.axis_index("s") * NC + lax.axis_index("c")
    base = wid * BPW

    pltpu.sync_copy(yaw_hbm.at[pl.ds(base, BPW)], yaw_v)
    pltpu.sync_copy(pitch_hbm.at[pl.ds(base, BPW)], pitch_v)
    pltpu.sync_copy(params_hbm, params_v)

    minx = params_v[pl.ds(0, L)]
    maxx = params_v[pl.ds(16, L)]
    intx = params_v[pl.ds(32, L)]
    miny = params_v[pl.ds(48, L)]
    maxy = params_v[pl.ds(64, L)]
    inty = params_v[pl.ds(80, L)]
    nx_i = params_v[pl.ds(96, L)].astype(jnp.int32)

    for j in range(BPW // L):
        yv = yaw_v[pl.ds(j * L, L)]
        pv = pitch_v[pl.ds(j * L, L)]
        y = yv - HALF_PI
        p = HALF_PI - pv
        xc = jnp.clip(y, minx, maxx)
        x_id = ((xc - minx) / intx + np.float32(0.5)).astype(jnp.int32)
        yc = jnp.clip(p, miny, maxy)
        y_id = ((yc - miny) / inty + np.float32(0.5)).astype(jnp.int32)
        idx_v[pl.ds(j * L, L)] = y_id * nx_i + x_id

    pltpu.sync_copy(idx_v, idx_out.at[pl.ds(base, BPW)])

    idx_vecs = [idx_v[pl.ds(j * L, L)] for j in range(BPW // L)]

    NT = 2 * BPW
    LEAD = CH // 2

    def src_for(t):
        i = idx_vecs[(t // 2) // L][(t // 2) % L]
        return (vids_hbm if t % 2 == 0 else wets_hbm).at[i]

    def dst_for(t):
        return (vids_out if t % 2 == 0 else wets_out).at[base + t // 2]

    g = {}
    wb = {}
    for t in range(NT + LEAD):
        if t >= LEAD:
            u = t - LEAD
            g[u].wait()
            wb[u] = pltpu.async_copy(buf.at[u % CH], dst_for(u), wsem)
        if t < NT:
            if t >= CH:
                wb[t - CH].wait()
            g[t] = pltpu.async_copy(src_for(t), buf.at[t % CH], gsem)
    for t in range(NT - CH, NT):
        wb[t].wait()


def kernel(yaw, pitch, n_y_p, min_v, max_v, vids, wets):
    vids2 = vids.reshape(T, D)
    wets2 = wets.reshape(T, D).view(jnp.int32)
    interval = (max_v - min_v) / (n_y_p - 1.0)
    params = jnp.concatenate([
        jnp.broadcast_to(min_v[0], (16,)),
        jnp.broadcast_to(max_v[0], (16,)),
        jnp.broadcast_to(interval[0], (16,)),
        jnp.broadcast_to(min_v[1], (16,)),
        jnp.broadcast_to(max_v[1], (16,)),
        jnp.broadcast_to(interval[1], (16,)),
        jnp.broadcast_to(n_y_p[0], (16,)),
        jnp.zeros((16,), jnp.float32),
    ])
    vids_rows, wets_rows, idx = _lookup(yaw, pitch, params, vids2, wets2)
    return (vids_rows.reshape(B, 3060, 3),
            wets_rows.view(jnp.float32).reshape(B, 3060, 3),
            idx)


# speedup vs baseline: 9.5903x; 2.3669x over previous
import functools

import jax
import jax.numpy as jnp
import numpy as np
from jax import lax
from jax.experimental import pallas as pl
from jax.experimental.pallas import tpu as pltpu
from jax.experimental.pallas import tpu_sc as plsc

B = 1024
T = 441
N_LDMK = 3060
N_BARY = 3
NCH = 24

NC, NS, L = 2, 16, 16
NW = NC * NS
BPW = B // NW
RING = 4
HALF_PI = np.float32(np.pi / 2)

_mesh = plsc.VectorSubcoreMesh(core_axis_name="c", subcore_axis_name="s")


@functools.partial(
    pl.kernel,
    out_type=(
        jax.ShapeDtypeStruct((B, NCH, 4, 128), jnp.int32),
        jax.ShapeDtypeStruct((B, NCH, 4, 128), jnp.float32),
        jax.ShapeDtypeStruct((B,), jnp.int32),
    ),
    mesh=_mesh,
    scratch_types=[
        pltpu.VMEM((BPW,), jnp.float32),
        pltpu.VMEM((BPW,), jnp.float32),
        pltpu.VMEM((128,), jnp.float32),
        pltpu.VMEM((BPW,), jnp.int32),
        pltpu.VMEM((RING, N_BARY, NCH, 128), jnp.int32),
        pltpu.VMEM((RING, N_BARY, NCH, 128), jnp.float32),
        pltpu.SemaphoreType.DMA,
        pltpu.SemaphoreType.DMA,
    ],
)
def _lookup(yaw_hbm, pitch_hbm, params_hbm, vids_hbm, wets_hbm,
            vids_out, wets_out, idx_out,
            yaw_v, pitch_v, params_v, idx_v, vbuf, wbuf, gsem, wsem):
    wid = lax.axis_index("s") * NC + lax.axis_index("c")
    base = wid * BPW

    pltpu.sync_copy(yaw_hbm.at[pl.ds(base, BPW)], yaw_v)
    pltpu.sync_copy(pitch_hbm.at[pl.ds(base, BPW)], pitch_v)
    pltpu.sync_copy(params_hbm, params_v)

    minx = params_v[pl.ds(0, L)]
    maxx = params_v[pl.ds(16, L)]
    intx = params_v[pl.ds(32, L)]
    miny = params_v[pl.ds(48, L)]
    maxy = params_v[pl.ds(64, L)]
    inty = params_v[pl.ds(80, L)]
    nx_i = params_v[pl.ds(96, L)].astype(jnp.int32)

    for j in range(BPW // L):
        yv = yaw_v[pl.ds(j * L, L)]
        pv = pitch_v[pl.ds(j * L, L)]
        y = yv - HALF_PI
        p = HALF_PI - pv
        xc = jnp.clip(y, minx, maxx)
        x_id = ((xc - minx) / intx + np.float32(0.5)).astype(jnp.int32)
        yc = jnp.clip(p, miny, maxy)
        y_id = ((yc - miny) / inty + np.float32(0.5)).astype(jnp.int32)
        idx_v[pl.ds(j * L, L)] = y_id * nx_i + x_id

    pltpu.sync_copy(idx_v, idx_out.at[pl.ds(base, BPW)])

    idx_vecs = [idx_v[pl.ds(j * L, L)] for j in range(BPW // L)]

    NT = 2 * BPW
    LEAD = RING

    def slot_for(t):
        return (vbuf if t % 2 == 0 else wbuf).at[(t // 2) % RING]

    def src_for(t):
        i = idx_vecs[(t // 2) // L][(t // 2) % L]
        return (vids_hbm if t % 2 == 0 else wets_hbm).at[i]

    def writeback(t):
        out = vids_out if t % 2 == 0 else wets_out
        slot = slot_for(t)
        return [
            pltpu.async_copy(slot.at[k], out.at[base + t // 2, :, k], wsem)
            for k in range(N_BARY)
        ]

    g = {}
    wb = {}
    for t in range(NT + LEAD):
        if t >= LEAD:
            u = t - LEAD
            g[u].wait()
            wb[u] = writeback(u)
        if t < NT:
            if t >= 2 * RING:
                for c in wb[t - 2 * RING]:
                    c.wait()
            g[t] = pltpu.async_copy(src_for(t), slot_for(t), gsem)
    for t in range(NT - 2 * RING, NT):
        for c in wb[t]:
            c.wait()


def _pack(tbl):
    x = tbl.reshape(T, N_LDMK, N_BARY).transpose(0, 2, 1)
    x = jnp.pad(x, ((0, 0), (0, 0), (0, NCH * 128 - N_LDMK)))
    return x.reshape(T, N_BARY, NCH, 128)


def _unpack(o):
    x = o.transpose(0, 1, 3, 2).reshape(B, NCH * 128, 4)
    return x[:, :N_LDMK, :N_BARY]


def kernel(yaw, pitch, n_y_p, min_v, max_v, vids, wets):
    interval = (max_v - min_v) / (n_y_p - 1.0)
    params = jnp.concatenate([
        jnp.broadcast_to(min_v[0], (16,)),
        jnp.broadcast_to(max_v[0], (16,)),
        jnp.broadcast_to(interval[0], (16,)),
        jnp.broadcast_to(min_v[1], (16,)),
        jnp.broadcast_to(max_v[1], (16,)),
        jnp.broadcast_to(interval[1], (16,)),
        jnp.broadcast_to(n_y_p[0], (16,)),
        jnp.zeros((16,), jnp.float32),
    ])
    vids_o, wets_o, idx = _lookup(yaw, pitch, params, _pack(vids), _pack(wets))
    return (_unpack(vids_o), _unpack(wets_o), idx)


# speedup vs baseline: 15.4488x; 1.6109x over previous
import functools

import jax
import jax.numpy as jnp
import numpy as np
from jax import lax
from jax.experimental import pallas as pl
from jax.experimental.pallas import tpu as pltpu
from jax.experimental.pallas import tpu_sc as plsc

B = 1024
T = 441
N_LDMK = 3060
N_BARY = 3
NC, NS, L = 2, 16, 16
NW = NC * NS
BPW = B // NW
RING = 2
HALF_PI = np.float32(np.pi / 2)

_mesh = plsc.VectorSubcoreMesh(core_axis_name="c", subcore_axis_name="s")


@functools.partial(
    pl.kernel,
    out_type=(
        jax.ShapeDtypeStruct((N_BARY, B, N_LDMK), jnp.int32),
        jax.ShapeDtypeStruct((N_BARY, B, N_LDMK), jnp.float32),
        jax.ShapeDtypeStruct((B,), jnp.int32),
    ),
    mesh=_mesh,
    scratch_types=[
        pltpu.VMEM((BPW,), jnp.float32),
        pltpu.VMEM((BPW,), jnp.float32),
        pltpu.VMEM((128,), jnp.float32),
        pltpu.VMEM((BPW,), jnp.int32),
        pltpu.VMEM((RING, N_BARY, N_LDMK), jnp.int32),
        pltpu.VMEM((RING, N_BARY, N_LDMK), jnp.float32),
        pltpu.SemaphoreType.DMA,
        pltpu.SemaphoreType.DMA,
    ],
)
def _lookup(yaw_hbm, pitch_hbm, params_hbm, vids_hbm, wets_hbm,
            vids_out, wets_out, idx_out,
            yaw_v, pitch_v, params_v, idx_v, vbuf, wbuf, gsem, wsem):
    wid = lax.axis_index("s") * NC + lax.axis_index("c")
    base = wid * BPW

    pltpu.sync_copy(yaw_hbm.at[pl.ds(base, BPW)], yaw_v)
    pltpu.sync_copy(pitch_hbm.at[pl.ds(base, BPW)], pitch_v)
    pltpu.sync_copy(params_hbm, params_v)

    minx = params_v[pl.ds(0, L)]
    maxx = params_v[pl.ds(16, L)]
    intx = params_v[pl.ds(32, L)]
    miny = params_v[pl.ds(48, L)]
    maxy = params_v[pl.ds(64, L)]
    inty = params_v[pl.ds(80, L)]
    nx_i = params_v[pl.ds(96, L)].astype(jnp.int32)

    for j in range(BPW // L):
        yv = yaw_v[pl.ds(j * L, L)]
        pv = pitch_v[pl.ds(j * L, L)]
        y = yv - HALF_PI
        p = HALF_PI - pv
        xc = jnp.clip(y, minx, maxx)
        x_id = ((xc - minx) / intx + np.float32(0.5)).astype(jnp.int32)
        yc = jnp.clip(p, miny, maxy)
        y_id = ((yc - miny) / inty + np.float32(0.5)).astype(jnp.int32)
        idx_v[pl.ds(j * L, L)] = y_id * nx_i + x_id

    pltpu.sync_copy(idx_v, idx_out.at[pl.ds(base, BPW)])

    idx_vecs = [idx_v[pl.ds(j * L, L)] for j in range(BPW // L)]

    NT = 2 * BPW
    LEAD = RING

    def slot_for(t):
        return (vbuf if t % 2 == 0 else wbuf).at[(t // 2) % RING]

    def gather(t):
        i = idx_vecs[(t // 2) // L][(t // 2) % L]
        table = vids_hbm if t % 2 == 0 else wets_hbm
        return pltpu.async_copy(table.at[:, i], slot_for(t), gsem)

    def writeback(t):
        out = vids_out if t % 2 == 0 else wets_out
        return pltpu.async_copy(slot_for(t), out.at[:, base + t // 2], wsem)

    g = {}
    wb = {}
    for t in range(NT + LEAD):
        if t >= LEAD:
            u = t - LEAD
            g[u].wait()
            wb[u] = writeback(u)
        if t < NT:
            if t >= 2 * RING:
                wb[t - 2 * RING].wait()
            g[t] = gather(t)
    for t in range(NT - 2 * RING, NT):
        wb[t].wait()


def kernel(yaw, pitch, n_y_p, min_v, max_v, vids, wets):
    interval = (max_v - min_v) / (n_y_p - 1.0)
    params = jnp.concatenate([
        jnp.broadcast_to(min_v[0], (16,)),
        jnp.broadcast_to(max_v[0], (16,)),
        jnp.broadcast_to(interval[0], (16,)),
        jnp.broadcast_to(min_v[1], (16,)),
        jnp.broadcast_to(max_v[1], (16,)),
        jnp.broadcast_to(interval[1], (16,)),
        jnp.broadcast_to(n_y_p[0], (16,)),
        jnp.zeros((16,), jnp.float32),
    ])
    vt = vids.reshape(T, N_LDMK, N_BARY).transpose(2, 0, 1)
    wt = wets.reshape(T, N_LDMK, N_BARY).transpose(2, 0, 1)
    vids_o, wets_o, idx = _lookup(yaw, pitch, params, vt, wt)
    return (vids_o.transpose(1, 2, 0), wets_o.transpose(1, 2, 0), idx)
